# baseline (speedup 1.0000x reference)
import functools

import jax
import jax.numpy as jnp
from jax import lax
from jax.experimental import pallas as pl
from jax.experimental.pallas import tpu as pltpu
from jax.experimental.pallas import tpu_sc as plsc

N_BASES = 8
TAU_MIN = 0.01
TAU_MAX = 10.0
EPS = 1e-5
N_LAYERS = 2

_DENOM = 4.0 / 7.0
_INV_D = 1.0 / _DENOM
_CENTERS = [-2.0 + k * _DENOM for k in range(N_BASES)]

_NC = 2
_NS = 16
_NW = _NC * _NS

_CH = 128


def _ceil_to(x, m):
    return (x + m - 1) // m * m


def _zero_fill(buf, width):
    z = jnp.zeros((16,), jnp.float32)

    @pl.loop(0, buf.shape[0])
    def _(r):
        for j in range(width // 16):
            buf[r, pl.ds(j * 16, 16)] = z


_NBUF = 4


def _make_agg_kernel(np_rows, chunk, width, n0, n1, pchunk):
    mesh = plsc.VectorSubcoreMesh(core_axis_name="c", subcore_axis_name="s")
    rows_per_sub = np_rows // _NS

    @functools.partial(
        pl.kernel,
        mesh=mesh,
        out_type=jax.ShapeDtypeStruct((_NC, np_rows, width), jnp.float32),
        scratch_types=[
            pltpu.VMEM((pchunk, chunk), jnp.int32),
            pltpu.VMEM((pchunk, chunk), jnp.int32),
        ] + [pltpu.VMEM((chunk, width), jnp.float32)] * _NBUF + [
            pltpu.VMEM_SHARED((np_rows, width), jnp.float32),
        ] + [pltpu.SemaphoreType.DMA] * (_NBUF + 1),
    )
    def agg_kernel(h_hbm, src_hbm, dst_hbm, out_hbm, src_v, dst_v, r0, r1,
                   r2, r3, acc, s0, s1, s2, s3, zsem):
        rows = [r0, r1, r2, r3]
        sems = [s0, s1, s2, s3]
        cid = lax.axis_index("c")
        sid = lax.axis_index("s")
        for b in range(_NBUF):
            _zero_fill(rows[b], width)
        base = sid * rows_per_sub
        nz = rows_per_sub // chunk
        nzb = [0] * _NBUF
        for i in range(nz):
            b = i % _NBUF
            pltpu.async_copy(rows[b], acc.at[pl.ds(base + i * chunk, chunk)],
                             sems[b])
            nzb[b] += 1
        for b in range(_NBUF):
            for _i in range(nzb[b]):
                pltpu.make_async_copy(rows[b], acc.at[pl.ds(base, chunk)],
                                      sems[b]).wait()
        plsc.subcore_barrier()

        n_c = jnp.where(cid == 0, n0, n1)
        cbase = jnp.where(cid == 0, sid * n0, _NS * n0 + sid * n1)

        @pl.loop(0, n_c // pchunk)
        def _(ph):
            pb = cbase + ph * pchunk
            i0 = pltpu.async_copy(src_hbm.at[pl.ds(pb, pchunk)], src_v, zsem)
            i1 = pltpu.async_copy(dst_hbm.at[pl.ds(pb, pchunk)], dst_v, zsem)
            i0.wait()
            i1.wait()
            for b in range(_NBUF):
                pltpu.async_copy(h_hbm.at[src_v.at[b]], rows[b], sems[b])

            @pl.loop(0, pchunk // _NBUF - 1)
            def _(r):
                j0 = r * _NBUF
                for b in range(_NBUF):
                    pltpu.make_async_copy(h_hbm.at[src_v.at[0]], rows[b],
                                          sems[b]).wait()
                    pltpu.async_copy(rows[b], acc.at[dst_v.at[j0 + b]],
                                     sems[b], add=True)
                for b in range(_NBUF):
                    pltpu.make_async_copy(h_hbm.at[src_v.at[0]], rows[b],
                                          sems[b]).wait()
                    pltpu.async_copy(h_hbm.at[src_v.at[j0 + _NBUF + b]],
                                     rows[b], sems[b])

            for b in range(_NBUF):
                pltpu.make_async_copy(h_hbm.at[src_v.at[0]], rows[b],
                                      sems[b]).wait()
                pltpu.async_copy(rows[b], acc.at[dst_v.at[pchunk - _NBUF + b]],
                                 sems[b], add=True)
            for b in range(_NBUF):
                pltpu.make_async_copy(h_hbm.at[src_v.at[0]], rows[b],
                                      sems[b]).wait()

        plsc.subcore_barrier()
        pltpu.sync_copy(
            acc.at[pl.ds(base, rows_per_sub)],
            out_hbm.at[cid, pl.ds(base, rows_per_sub)],
        )

    return agg_kernel


def _kan_body(x_ref, w_ref, b_ref, o_ref):
    xb = x_ref[...]
    acc = None
    for k in range(N_BASES):
        phi = jnp.exp(-(((xb - _CENTERS[k]) * _INV_D) ** 2))
        part = jnp.dot(phi.astype(jnp.bfloat16), w_ref[k],
                       preferred_element_type=jnp.float32)
        acc = part if acc is None else acc + part
    o_ref[...] = acc + b_ref[...]


def _kan_call(x, w8, b, block_rows):
    rows, din = x.shape
    dout = w8.shape[-1]
    grid = rows // block_rows
    return pl.pallas_call(
        _kan_body,
        grid=(grid,),
        in_specs=[
            pl.BlockSpec((block_rows, din), lambda i: (i, 0)),
            pl.BlockSpec((N_BASES, din, dout), lambda i: (0, 0, 0)),
            pl.BlockSpec((1, dout), lambda i: (0, 0)),
        ],
        out_specs=pl.BlockSpec((block_rows, dout), lambda i: (i, 0)),
        out_shape=jax.ShapeDtypeStruct((rows, dout), jnp.float32),
    )(x, w8, b)


def _cell_body(cin_ref, h_ref, p_ref, invd_ref, wt_ref, bt_ref, wk_ref,
               bk_ref, g_ref, be_ref, o_ref, *, residual, dt, lane_block):
    cin = cin_ref[...]
    h = h_ref[...]
    hdim = h.shape[-1]
    lo = lane_block * hdim
    psum = p_ref[0] + p_ref[1]
    m = psum[:, lo:lo + hdim] * invd_ref[...]
    pre = jnp.concatenate([cin, h, m], axis=-1)
    tau_lin = jnp.dot(pre.astype(jnp.bfloat16), wt_ref[...],
                      preferred_element_type=jnp.float32)
    tau = TAU_MIN + (TAU_MAX - TAU_MIN) * jax.nn.sigmoid(tau_lin + bt_ref[...])
    acc = None
    for k in range(N_BASES):
        phi = jnp.exp(-(((pre - _CENTERS[k]) * _INV_D) ** 2))
        part = jnp.dot(phi.astype(jnp.bfloat16), wk_ref[k],
                       preferred_element_type=jnp.float32)
        acc = part if acc is None else acc + part
    h_tgt = jnp.tanh(acc + bk_ref[...])
    h_new = h + dt * (h_tgt - h) / tau
    mu = jnp.mean(h_new, axis=-1, keepdims=True)
    var = jnp.mean((h_new - mu) ** 2, axis=-1, keepdims=True)
    y = (h_new - mu) * lax.rsqrt(var + EPS) * g_ref[...] + be_ref[...]
    if residual:
        y = y + h
    o_ref[...] = y


def _cell_call(cin, cin_block_off, h, part, lane_block, inv_deg,
               wt, bt, wk8, bk, g, be, residual, dt, n, hdim, block_rows):
    grid = n // block_rows
    pw = part.shape[-1]
    body = functools.partial(_cell_body, residual=residual, dt=dt,
                             lane_block=lane_block)
    return pl.pallas_call(
        body,
        grid=(grid,),
        in_specs=[
            pl.BlockSpec((block_rows, hdim),
                         lambda i, o=cin_block_off: (o + i, 0)),
            pl.BlockSpec((block_rows, hdim), lambda i: (i, 0)),
            pl.BlockSpec((_NC, block_rows, pw), lambda i: (0, i, 0)),
            pl.BlockSpec((block_rows, 1), lambda i: (i, 0)),
            pl.BlockSpec((3 * hdim, hdim), lambda i: (0, 0)),
            pl.BlockSpec((1, hdim), lambda i: (0, 0)),
            pl.BlockSpec((N_BASES, 3 * hdim, hdim), lambda i: (0, 0, 0)),
            pl.BlockSpec((1, hdim), lambda i: (0, 0)),
            pl.BlockSpec((1, hdim), lambda i: (0, 0)),
            pl.BlockSpec((1, hdim), lambda i: (0, 0)),
        ],
        out_specs=pl.BlockSpec((block_rows, hdim), lambda i: (i, 0)),
        out_shape=jax.ShapeDtypeStruct((n, hdim), jnp.float32),
    )(cin, h, part, inv_deg, wt, bt, wk8, bk, g, be)


def kernel(x, edge_index, W_enc, b_enc, W_tau0, b_tau0, W_kan0, b_kan0, g0,
           be0, W_tau1, b_tau1, W_kan1, b_kan1, g1, be1, W_dec, b_dec, h0):
    T, N, F = x.shape
    H = h0.shape[-1]
    O = W_dec.shape[-1]
    E = edge_index.shape[1]
    dt = 1.0 / T

    bf = jnp.bfloat16
    w_enc8 = W_enc.reshape(F, N_BASES, H).transpose(1, 0, 2).astype(bf)
    wk0 = W_kan0.reshape(3 * H, N_BASES, H).transpose(1, 0, 2).astype(bf)
    wk1 = W_kan1.reshape(3 * H, N_BASES, H).transpose(1, 0, 2).astype(bf)
    wt0 = W_tau0.astype(bf)
    wt1 = W_tau1.astype(bf)
    w_dec8 = W_dec.reshape(H, N_BASES, O).transpose(1, 0, 2)
    w_dec8 = jnp.pad(w_dec8, ((0, 0), (0, 0), (0, 128 - O))).astype(bf)
    b_decp = jnp.pad(b_dec.reshape(1, O), ((0, 0), (0, 128 - O)))

    ch_agg = 40
    ep = _ceil_to(E, _NS * ch_agg * 2 * _NBUF)
    tot_chunks = ep // ch_agg
    n_per = tot_chunks // _NS
    n0 = max(_NBUF, (7 * n_per // 8) // _NBUF * _NBUF)
    pchunk = 32 if (n0 % 32 == 0 and (n_per - n0) % 32 == 0) else _NBUF
    n1 = n_per - n0
    np_rows = _ceil_to(N + 1, 16 * _NS)

    src = edge_index[0]
    dst = edge_index[1]
    pad = ep - E
    src_flat = jnp.concatenate([src, jnp.zeros((pad,), jnp.int32)])
    dst_flat = jnp.concatenate([dst, jnp.full((pad,), N, jnp.int32)])
    srcp = src_flat.reshape(tot_chunks, ch_agg)
    dstp = dst_flat.reshape(tot_chunks, ch_agg)

    agg_kernel = _make_agg_kernel(np_rows, ch_agg, 2 * H, n0, n1, pchunk)

    x0 = jnp.broadcast_to(
        jnp.concatenate([h0, jnp.ones((H,), jnp.float32)]), (N, 2 * H))
    degp = agg_kernel(x0, srcp, dstp)
    deg = degp[0, :N, H] + degp[1, :N, H]
    inv_deg = (1.0 / jnp.maximum(deg, 1.0)).reshape(N, 1)

    u_all = _kan_call(x.reshape(T * N, F), w_enc8, b_enc.reshape(1, H), 1000)

    blk = 1000
    nblk = N // blk
    h_l0 = jnp.broadcast_to(h0, (N, H))
    h_l1 = h_l0
    h1_steps = []
    for t in range(T):
        if t == 0:
            p = degp
            lb0, lb1 = 0, 0
        else:
            hcat = jnp.concatenate([h_l0, h_l1], axis=1)
            p = agg_kernel(hcat, srcp, dstp)
            lb0, lb1 = 0, 1
        h_l0 = _cell_call(u_all, t * nblk, h_l0, p, lb0, inv_deg,
                          wt0, b_tau0.reshape(1, H), wk0,
                          b_kan0.reshape(1, H), g0.reshape(1, H),
                          be0.reshape(1, H), False, dt, N, H, blk)
        h_l1 = _cell_call(h_l0, 0, h_l1, p, lb1, inv_deg,
                          wt1, b_tau1.reshape(1, H), wk1,
                          b_kan1.reshape(1, H), g1.reshape(1, H),
                          be1.reshape(1, H), True, dt, N, H, blk)
        h1_steps.append(h_l1)

    hstack = jnp.concatenate(h1_steps, axis=0)
    dec = _kan_call(hstack, w_dec8, b_decp, 1000)
    return dec[:, :O].reshape(T, N, O)


# speedup vs baseline: 1.0306x; 1.0306x over previous
import functools

import jax
import jax.numpy as jnp
from jax import lax
from jax.experimental import pallas as pl
from jax.experimental.pallas import tpu as pltpu
from jax.experimental.pallas import tpu_sc as plsc

N_BASES = 8
TAU_MIN = 0.01
TAU_MAX = 10.0
EPS = 1e-5
N_LAYERS = 2

_DENOM = 4.0 / 7.0
_INV_D = 1.0 / _DENOM
_CENTERS = [-2.0 + k * _DENOM for k in range(N_BASES)]

_NC = 2
_NS = 16
_NW = _NC * _NS

_CH = 128


def _ceil_to(x, m):
    return (x + m - 1) // m * m


def _zero_fill(buf, width):
    z = jnp.zeros((16,), jnp.float32)

    @pl.loop(0, buf.shape[0])
    def _(r):
        for j in range(width // 16):
            buf[r, pl.ds(j * 16, 16)] = z


_NBUF = 4


def _make_agg_kernel(np_rows, chunk, width, n0, n1, pchunk):
    mesh = plsc.VectorSubcoreMesh(core_axis_name="c", subcore_axis_name="s")
    rows_per_sub = np_rows // _NS

    @functools.partial(
        pl.kernel,
        mesh=mesh,
        out_type=jax.ShapeDtypeStruct((_NC, np_rows, width), jnp.float32),
        scratch_types=[
            pltpu.VMEM((pchunk, chunk), jnp.int32),
            pltpu.VMEM((pchunk, chunk), jnp.int32),
        ] + [pltpu.VMEM((chunk, width), jnp.float32)] * _NBUF + [
            pltpu.VMEM_SHARED((np_rows, width), jnp.float32),
        ] + [pltpu.SemaphoreType.DMA] * (_NBUF + 1),
    )
    def agg_kernel(h_hbm, src_hbm, dst_hbm, out_hbm, src_v, dst_v, r0, r1,
                   r2, r3, acc, s0, s1, s2, s3, zsem):
        rows = [r0, r1, r2, r3]
        sems = [s0, s1, s2, s3]
        cid = lax.axis_index("c")
        sid = lax.axis_index("s")
        for b in range(_NBUF):
            _zero_fill(rows[b], width)
        base = sid * rows_per_sub
        nz = rows_per_sub // chunk
        nzb = [0] * _NBUF
        for i in range(nz):
            b = i % _NBUF
            pltpu.async_copy(rows[b], acc.at[pl.ds(base + i * chunk, chunk)],
                             sems[b])
            nzb[b] += 1
        for b in range(_NBUF):
            for _i in range(nzb[b]):
                pltpu.make_async_copy(rows[b], acc.at[pl.ds(base, chunk)],
                                      sems[b]).wait()
        plsc.subcore_barrier()

        n_c = jnp.where(cid == 0, n0, n1)
        cbase = jnp.where(cid == 0, sid * n0, _NS * n0 + sid * n1)

        @pl.loop(0, n_c // pchunk)
        def _(ph):
            pb = cbase + ph * pchunk
            i0 = pltpu.async_copy(src_hbm.at[pl.ds(pb, pchunk)], src_v, zsem)
            i1 = pltpu.async_copy(dst_hbm.at[pl.ds(pb, pchunk)], dst_v, zsem)
            i0.wait()
            i1.wait()
            for b in range(_NBUF):
                pltpu.async_copy(h_hbm.at[src_v.at[b]], rows[b], sems[b])

            @pl.loop(0, pchunk // _NBUF - 1)
            def _(r):
                j0 = r * _NBUF
                for b in range(_NBUF):
                    pltpu.make_async_copy(h_hbm.at[src_v.at[0]], rows[b],
                                          sems[b]).wait()
                    pltpu.async_copy(rows[b], acc.at[dst_v.at[j0 + b]],
                                     sems[b], add=True)
                for b in range(_NBUF):
                    pltpu.make_async_copy(h_hbm.at[src_v.at[0]], rows[b],
                                          sems[b]).wait()
                    pltpu.async_copy(h_hbm.at[src_v.at[j0 + _NBUF + b]],
                                     rows[b], sems[b])

            for b in range(_NBUF):
                pltpu.make_async_copy(h_hbm.at[src_v.at[0]], rows[b],
                                      sems[b]).wait()
                pltpu.async_copy(rows[b], acc.at[dst_v.at[pchunk - _NBUF + b]],
                                 sems[b], add=True)
            for b in range(_NBUF):
                pltpu.make_async_copy(h_hbm.at[src_v.at[0]], rows[b],
                                      sems[b]).wait()

        plsc.subcore_barrier()
        pltpu.sync_copy(
            acc.at[pl.ds(base, rows_per_sub)],
            out_hbm.at[cid, pl.ds(base, rows_per_sub)],
        )

    return agg_kernel


_ANCHOR = 3
_CK = [2.718281828459045 ** (-(_CENTERS[k + 1] + _CENTERS[k]) * _INV_D)
       for k in range(N_BASES - 1)]


def _phi_ladder(x):
    d_up = jnp.exp((2.0 * _INV_D) * x)
    d_dn = 1.0 / d_up
    phis = [None] * N_BASES
    phis[_ANCHOR] = jnp.exp(-(((x - _CENTERS[_ANCHOR]) * _INV_D) ** 2))
    for k in range(_ANCHOR, N_BASES - 1):
        phis[k + 1] = phis[k] * d_up * _CK[k]
    for k in range(_ANCHOR, 0, -1):
        phis[k - 1] = phis[k] * d_dn * (1.0 / _CK[k - 1])
    return phis


def _kan_body(x_ref, w_ref, b_ref, o_ref):
    xb = x_ref[...]
    phis = _phi_ladder(xb)
    acc = None
    for k in range(N_BASES):
        part = jnp.dot(phis[k].astype(jnp.bfloat16), w_ref[k],
                       preferred_element_type=jnp.float32)
        acc = part if acc is None else acc + part
    o_ref[...] = acc + b_ref[...]


def _kan_call(x, w8, b, block_rows):
    rows, din = x.shape
    dout = w8.shape[-1]
    grid = rows // block_rows
    return pl.pallas_call(
        _kan_body,
        grid=(grid,),
        in_specs=[
            pl.BlockSpec((block_rows, din), lambda i: (i, 0)),
            pl.BlockSpec((N_BASES, din, dout), lambda i: (0, 0, 0)),
            pl.BlockSpec((1, dout), lambda i: (0, 0)),
        ],
        out_specs=pl.BlockSpec((block_rows, dout), lambda i: (i, 0)),
        out_shape=jax.ShapeDtypeStruct((rows, dout), jnp.float32),
    )(x, w8, b)


def _cell_body(cin_ref, h_ref, p_ref, invd_ref, wt_ref, bt_ref, wk_ref,
               bk_ref, g_ref, be_ref, o_ref, *, residual, dt, lane_block):
    cin = cin_ref[...]
    h = h_ref[...]
    hdim = h.shape[-1]
    lo = lane_block * hdim
    psum = p_ref[0] + p_ref[1]
    m = psum[:, lo:lo + hdim] * invd_ref[...]
    pre = jnp.concatenate([cin, h, m], axis=-1)
    tau_lin = jnp.dot(pre.astype(jnp.bfloat16), wt_ref[...],
                      preferred_element_type=jnp.float32)
    tau = TAU_MIN + (TAU_MAX - TAU_MIN) * jax.nn.sigmoid(tau_lin + bt_ref[...])
    phis = _phi_ladder(pre)
    acc = None
    for k in range(N_BASES):
        part = jnp.dot(phis[k].astype(jnp.bfloat16), wk_ref[k],
                       preferred_element_type=jnp.float32)
        acc = part if acc is None else acc + part
    h_tgt = jnp.tanh(acc + bk_ref[...])
    h_new = h + dt * (h_tgt - h) / tau
    mu = jnp.mean(h_new, axis=-1, keepdims=True)
    var = jnp.mean((h_new - mu) ** 2, axis=-1, keepdims=True)
    y = (h_new - mu) * lax.rsqrt(var + EPS) * g_ref[...] + be_ref[...]
    if residual:
        y = y + h
    o_ref[...] = y


def _cell_call(cin, cin_block_off, h, part, lane_block, inv_deg,
               wt, bt, wk8, bk, g, be, residual, dt, n, hdim, block_rows):
    grid = n // block_rows
    pw = part.shape[-1]
    body = functools.partial(_cell_body, residual=residual, dt=dt,
                             lane_block=lane_block)
    return pl.pallas_call(
        body,
        grid=(grid,),
        in_specs=[
            pl.BlockSpec((block_rows, hdim),
                         lambda i, o=cin_block_off: (o + i, 0)),
            pl.BlockSpec((block_rows, hdim), lambda i: (i, 0)),
            pl.BlockSpec((_NC, block_rows, pw), lambda i: (0, i, 0)),
            pl.BlockSpec((block_rows, 1), lambda i: (i, 0)),
            pl.BlockSpec((3 * hdim, hdim), lambda i: (0, 0)),
            pl.BlockSpec((1, hdim), lambda i: (0, 0)),
            pl.BlockSpec((N_BASES, 3 * hdim, hdim), lambda i: (0, 0, 0)),
            pl.BlockSpec((1, hdim), lambda i: (0, 0)),
            pl.BlockSpec((1, hdim), lambda i: (0, 0)),
            pl.BlockSpec((1, hdim), lambda i: (0, 0)),
        ],
        out_specs=pl.BlockSpec((block_rows, hdim), lambda i: (i, 0)),
        out_shape=jax.ShapeDtypeStruct((n, hdim), jnp.float32),
    )(cin, h, part, inv_deg, wt, bt, wk8, bk, g, be)


def kernel(x, edge_index, W_enc, b_enc, W_tau0, b_tau0, W_kan0, b_kan0, g0,
           be0, W_tau1, b_tau1, W_kan1, b_kan1, g1, be1, W_dec, b_dec, h0):
    T, N, F = x.shape
    H = h0.shape[-1]
    O = W_dec.shape[-1]
    E = edge_index.shape[1]
    dt = 1.0 / T

    bf = jnp.bfloat16
    w_enc8 = W_enc.reshape(F, N_BASES, H).transpose(1, 0, 2).astype(bf)
    wk0 = W_kan0.reshape(3 * H, N_BASES, H).transpose(1, 0, 2).astype(bf)
    wk1 = W_kan1.reshape(3 * H, N_BASES, H).transpose(1, 0, 2).astype(bf)
    wt0 = W_tau0.astype(bf)
    wt1 = W_tau1.astype(bf)
    w_dec8 = W_dec.reshape(H, N_BASES, O).transpose(1, 0, 2)
    w_dec8 = jnp.pad(w_dec8, ((0, 0), (0, 0), (0, 128 - O))).astype(bf)
    b_decp = jnp.pad(b_dec.reshape(1, O), ((0, 0), (0, 128 - O)))

    ch_agg = 40
    ep = _ceil_to(E, _NS * ch_agg * 2 * _NBUF)
    tot_chunks = ep // ch_agg
    n_per = tot_chunks // _NS
    n0 = max(_NBUF, (7 * n_per // 8) // _NBUF * _NBUF)
    pchunk = 32 if (n0 % 32 == 0 and (n_per - n0) % 32 == 0) else _NBUF
    n1 = n_per - n0
    np_rows = _ceil_to(N + 1, 16 * _NS)

    src = edge_index[0]
    dst = edge_index[1]
    pad = ep - E
    src_flat = jnp.concatenate([src, jnp.zeros((pad,), jnp.int32)])
    dst_flat = jnp.concatenate([dst, jnp.full((pad,), N, jnp.int32)])
    srcp = src_flat.reshape(tot_chunks, ch_agg)
    dstp = dst_flat.reshape(tot_chunks, ch_agg)

    agg_kernel = _make_agg_kernel(np_rows, ch_agg, 2 * H, n0, n1, pchunk)

    x0 = jnp.broadcast_to(
        jnp.concatenate([h0, jnp.ones((H,), jnp.float32)]), (N, 2 * H))
    degp = agg_kernel(x0, srcp, dstp)
    deg = degp[0, :N, H] + degp[1, :N, H]
    inv_deg = (1.0 / jnp.maximum(deg, 1.0)).reshape(N, 1)

    u_all = _kan_call(x.reshape(T * N, F), w_enc8, b_enc.reshape(1, H), 1000)

    blk = 1000
    nblk = N // blk
    h_l0 = jnp.broadcast_to(h0, (N, H))
    h_l1 = h_l0
    h1_steps = []
    for t in range(T):
        if t == 0:
            p = degp
            lb0, lb1 = 0, 0
        else:
            hcat = jnp.concatenate([h_l0, h_l1], axis=1)
            p = agg_kernel(hcat, srcp, dstp)
            lb0, lb1 = 0, 1
        h_l0 = _cell_call(u_all, t * nblk, h_l0, p, lb0, inv_deg,
                          wt0, b_tau0.reshape(1, H), wk0,
                          b_kan0.reshape(1, H), g0.reshape(1, H),
                          be0.reshape(1, H), False, dt, N, H, blk)
        h_l1 = _cell_call(h_l0, 0, h_l1, p, lb1, inv_deg,
                          wt1, b_tau1.reshape(1, H), wk1,
                          b_kan1.reshape(1, H), g1.reshape(1, H),
                          be1.reshape(1, H), True, dt, N, H, blk)
        h1_steps.append(h_l1)

    hstack = jnp.concatenate(h1_steps, axis=0)
    dec = _kan_call(hstack, w_dec8, b_decp, 1000)
    return dec[:, :O].reshape(T, N, O)
